# baseline (speedup 1.0000x reference)
import functools

import jax
import jax.numpy as jnp
from jax import lax
from jax.experimental import pallas as pl
from jax.experimental.pallas import tpu as pltpu
from jax.experimental.pallas import tpu_sc as plsc

N = 10000
D = 128
E = 320000
STEPS = 4

N_PAD = 10240
PAD_ROW = N
NC, NS = 2, 16
NW = NC * NS
E_PAD = 327680
EPT = E_PAD // NW
CHUNK = 128
NCHUNK = EPT // CHUNK
RPT = N_PAD // NS
DEG_W = 128

_mesh = plsc.VectorSubcoreMesh(core_axis_name="c", subcore_axis_name="s")


@functools.partial(
    pl.kernel,
    mesh=_mesh,
    out_type=jax.ShapeDtypeStruct((NC * N_PAD, DEG_W), jnp.float32),
    scratch_types=[
        pltpu.VMEM((NCHUNK, CHUNK), jnp.int32),
        pltpu.VMEM((CHUNK, DEG_W), jnp.float32),
        pltpu.VMEM((16, DEG_W), jnp.float32),
        pltpu.VMEM_SHARED((N_PAD, DEG_W), jnp.float32),
        pltpu.SemaphoreType.DMA,
    ],
)
def _sc_degree(dst_hbm, out_hbm, idx_d, ones_v, zb, acc, sem):
    cid = lax.axis_index("c")
    sid = lax.axis_index("s")
    wid = cid * NS + sid

    @pl.loop(0, CHUNK)
    def _(r):
        @pl.loop(0, DEG_W, step=16)
        def _(j):
            ones_v[r, pl.ds(j, 16)] = jnp.ones((16,), jnp.float32)

    @pl.loop(0, 16)
    def _(r):
        @pl.loop(0, DEG_W, step=16)
        def _(j):
            zb[r, pl.ds(j, 16)] = jnp.zeros((16,), jnp.float32)

    @pl.loop(0, RPT // 16)
    def _(b):
        pltpu.sync_copy(zb, acc.at[pl.ds(sid * RPT + b * 16, 16)])

    pltpu.async_copy(dst_hbm.at[wid], idx_d, sem).wait()
    plsc.subcore_barrier()

    @pl.loop(0, NCHUNK)
    def _(c):
        pltpu.sync_copy(ones_v, acc.at[idx_d.at[c]], add=True)

    plsc.subcore_barrier()
    pltpu.sync_copy(
        acc.at[pl.ds(sid * RPT, RPT)],
        out_hbm.at[pl.ds(cid * N_PAD + sid * RPT, RPT)],
    )


PH = 4
CPP = NCHUNK // PH


def _adj_phase(hp_hbm, cs, cd, buf_a, buf_b, acc, sem_a, sem_b):
    pltpu.async_copy(hp_hbm.at[cs.at[0]], buf_a, sem_a)
    pltpu.async_copy(hp_hbm.at[cs.at[1]], buf_b, sem_b)

    @pl.loop(0, CPP // 2)
    def _(p):
        c = p * 2
        pltpu.make_async_copy(hp_hbm.at[cs.at[c]], buf_a, sem_a).wait()
        pltpu.sync_copy(buf_a, acc.at[cd.at[c]], add=True)

        @pl.when(p < CPP // 2 - 1)
        def _():
            pltpu.async_copy(hp_hbm.at[cs.at[c + 2]], buf_a, sem_a)

        pltpu.make_async_copy(hp_hbm.at[cs.at[c + 1]], buf_b, sem_b).wait()
        pltpu.sync_copy(buf_b, acc.at[cd.at[c + 1]], add=True)

        @pl.when(p < CPP // 2 - 1)
        def _():
            pltpu.async_copy(hp_hbm.at[cs.at[c + 3]], buf_b, sem_b)


@functools.partial(
    pl.kernel,
    mesh=_mesh,
    out_type=jax.ShapeDtypeStruct((NC * N_PAD, D), jnp.float32),
    scratch_types=[
        pltpu.VMEM((CPP, CHUNK), jnp.int32),
        pltpu.VMEM((CPP, CHUNK), jnp.int32),
        pltpu.VMEM((CPP, CHUNK), jnp.int32),
        pltpu.VMEM((CPP, CHUNK), jnp.int32),
        pltpu.VMEM((CHUNK, D), jnp.float32),
        pltpu.VMEM((CHUNK, D), jnp.float32),
        pltpu.VMEM_SHARED((N_PAD, D), jnp.float32),
        pltpu.SemaphoreType.DMA,
        pltpu.SemaphoreType.DMA,
        pltpu.SemaphoreType.DMA,
        pltpu.SemaphoreType.DMA,
    ],
)
def _sc_adj(hp_hbm, src_hbm, dst_hbm, zeros_hbm, out_hbm, idx_s0, idx_s1,
            idx_d0, idx_d1, buf_a, buf_b, acc, sem_i, sem_a, sem_b, sem_z):
    cid = lax.axis_index("c")
    sid = lax.axis_index("s")
    wid = cid * NS + sid

    ic_s = pltpu.async_copy(src_hbm.at[wid * PH], idx_s0, sem_i)
    ic_d = pltpu.async_copy(dst_hbm.at[wid * PH], idx_d0, sem_i)

    @pl.loop(0, RPT // 64)
    def _(b):
        pltpu.async_copy(zeros_hbm, acc.at[pl.ds(sid * RPT + b * 64, 64)],
                         sem_z)

    @pl.loop(0, RPT // 64)
    def _(b):
        pltpu.make_async_copy(zeros_hbm,
                              acc.at[pl.ds(sid * RPT + b * 64, 64)],
                              sem_z).wait()

    ic_s.wait()
    ic_d.wait()
    plsc.subcore_barrier()

    idx = [(idx_s0, idx_d0), (idx_s1, idx_d1)]
    for ph in range(PH):
        cs, cd = idx[ph % 2]
        ns, nd = idx[(ph + 1) % 2]
        if ph < PH - 1:
            pltpu.async_copy(src_hbm.at[wid * PH + ph + 1], ns, sem_i)
            pltpu.async_copy(dst_hbm.at[wid * PH + ph + 1], nd, sem_i)
        _adj_phase(hp_hbm, cs, cd, buf_a, buf_b, acc, sem_a, sem_b)
        if ph < PH - 1:
            pltpu.make_async_copy(src_hbm.at[wid * PH + ph + 1], ns,
                                  sem_i).wait()
            pltpu.make_async_copy(dst_hbm.at[wid * PH + ph + 1], nd,
                                  sem_i).wait()

    plsc.subcore_barrier()
    pltpu.sync_copy(
        acc.at[pl.ds(sid * RPT, RPT)],
        out_hbm.at[pl.ds(cid * N_PAD + sid * RPT, RPT)],
    )


_BLK = 256
_GRID = N_PAD // _BLK


def _tc_prep_body(d0_ref, d1_ref, x_ref, dis_ref, hp_ref):
    i = pl.program_id(0)
    row = i * _BLK + lax.broadcasted_iota(jnp.int32, (_BLK, 1), 0)
    deg = d0_ref[:, 0:1] + d1_ref[:, 0:1]
    dis = jnp.where(deg > 0, lax.rsqrt(jnp.where(deg > 0, deg, 1.0)), 0.0)
    dis = jnp.where(row < N, dis, 0.0)
    dis_b = jnp.broadcast_to(dis, (_BLK, D))
    dis_ref[...] = dis_b
    hp_ref[...] = dis_b * x_ref[...]


def _tc_prep(d0, d1, x_pad):
    return pl.pallas_call(
        _tc_prep_body,
        grid=(_GRID,),
        in_specs=[
            pl.BlockSpec((_BLK, DEG_W), lambda i: (i, 0)),
            pl.BlockSpec((_BLK, DEG_W), lambda i: (i, 0)),
            pl.BlockSpec((_BLK, D), lambda i: (i, 0)),
        ],
        out_specs=[
            pl.BlockSpec((_BLK, D), lambda i: (i, 0)),
            pl.BlockSpec((_BLK, D), lambda i: (i, 0)),
        ],
        out_shape=[
            jax.ShapeDtypeStruct((N_PAD, D), jnp.float32),
            jax.ShapeDtypeStruct((N_PAD, D), jnp.float32),
        ],
    )(d0, d1, x_pad)


def _tc_step_body(h_ref, p0_ref, p1_ref, dis_ref, w0_ref, w1_ref,
                  hn_ref, hpn_ref):
    hp = dis_ref[...] * (p0_ref[...] + p1_ref[...])
    hn = jnp.dot(h_ref[...], w0_ref[...],
                 preferred_element_type=jnp.float32,
                 precision=lax.Precision.HIGHEST)
    hn += jnp.dot(hp, w1_ref[...],
                  preferred_element_type=jnp.float32,
                  precision=lax.Precision.HIGHEST)
    hn_ref[...] = hn
    hpn_ref[...] = dis_ref[...] * hn


def _tc_step(h, p0, p1, dis_b, w0, w1):
    return pl.pallas_call(
        _tc_step_body,
        grid=(_GRID,),
        in_specs=[
            pl.BlockSpec((_BLK, D), lambda i: (i, 0)),
            pl.BlockSpec((_BLK, D), lambda i: (i, 0)),
            pl.BlockSpec((_BLK, D), lambda i: (i, 0)),
            pl.BlockSpec((_BLK, D), lambda i: (i, 0)),
            pl.BlockSpec((D, D), lambda i: (0, 0)),
            pl.BlockSpec((D, D), lambda i: (0, 0)),
        ],
        out_specs=[
            pl.BlockSpec((_BLK, D), lambda i: (i, 0)),
            pl.BlockSpec((_BLK, D), lambda i: (i, 0)),
        ],
        out_shape=[
            jax.ShapeDtypeStruct((N_PAD, D), jnp.float32),
            jax.ShapeDtypeStruct((N_PAD, D), jnp.float32),
        ],
    )(h, p0, p1, dis_b, w0, w1)


def kernel(x, edge_index, W):
    x_pad = jnp.pad(x, ((0, N_PAD - N), (0, 0)))
    src = jnp.pad(edge_index[0], (0, E_PAD - E), constant_values=PAD_ROW)
    dst = jnp.pad(edge_index[1], (0, E_PAD - E), constant_values=PAD_ROW)
    src4 = src.reshape(NW * PH, CPP, CHUNK)
    dst4 = dst.reshape(NW * PH, CPP, CHUNK)
    dst3 = dst.reshape(NW, NCHUNK, CHUNK)

    zblk = jnp.zeros((64, D), jnp.float32)
    deg = _sc_degree(dst3)
    dis_b, hp = _tc_prep(deg[:N_PAD], deg[N_PAD:], x_pad)

    h = x_pad
    for s in range(STEPS):
        parts = _sc_adj(hp, src4, dst4, zblk)
        h, hp = _tc_step(h, parts[:N_PAD], parts[N_PAD:], dis_b,
                         W[s, 0], W[s, 1])
    return h[:N]


# speedup vs baseline: 1.0391x; 1.0391x over previous
import functools

import jax
import jax.numpy as jnp
from jax import lax
from jax.experimental import pallas as pl
from jax.experimental.pallas import tpu as pltpu
from jax.experimental.pallas import tpu_sc as plsc

N = 10000
D = 128
E = 320000
STEPS = 4

N_PAD = 10240
PAD_ROW = N
NC, NS = 2, 16
NW = NC * NS
E_PAD = 327680
EPT = E_PAD // NW
CHUNK = 128
NCHUNK = EPT // CHUNK
RPT = N_PAD // NS
DEG_W = 128

_mesh = plsc.VectorSubcoreMesh(core_axis_name="c", subcore_axis_name="s")


@functools.partial(
    pl.kernel,
    mesh=_mesh,
    out_type=jax.ShapeDtypeStruct((NC * N_PAD, DEG_W), jnp.float32),
    scratch_types=[
        pltpu.VMEM((NCHUNK, CHUNK), jnp.int32),
        pltpu.VMEM((CHUNK, DEG_W), jnp.float32),
        pltpu.VMEM((16, DEG_W), jnp.float32),
        pltpu.VMEM_SHARED((N_PAD, DEG_W), jnp.float32),
        pltpu.SemaphoreType.DMA,
    ],
)
def _sc_degree(dst_hbm, out_hbm, idx_d, ones_v, zb, acc, sem):
    cid = lax.axis_index("c")
    sid = lax.axis_index("s")
    wid = cid * NS + sid

    @pl.loop(0, CHUNK)
    def _(r):
        @pl.loop(0, DEG_W, step=16)
        def _(j):
            ones_v[r, pl.ds(j, 16)] = jnp.ones((16,), jnp.float32)

    @pl.loop(0, 16)
    def _(r):
        @pl.loop(0, DEG_W, step=16)
        def _(j):
            zb[r, pl.ds(j, 16)] = jnp.zeros((16,), jnp.float32)

    @pl.loop(0, RPT // 16)
    def _(b):
        pltpu.sync_copy(zb, acc.at[pl.ds(sid * RPT + b * 16, 16)])

    pltpu.async_copy(dst_hbm.at[wid], idx_d, sem).wait()
    plsc.subcore_barrier()

    @pl.loop(0, NCHUNK)
    def _(c):
        pltpu.sync_copy(ones_v, acc.at[idx_d.at[c]], add=True)

    plsc.subcore_barrier()
    pltpu.sync_copy(
        acc.at[pl.ds(sid * RPT, RPT)],
        out_hbm.at[pl.ds(cid * N_PAD + sid * RPT, RPT)],
    )


PH = 4
CPP = NCHUNK // PH


def _adj_phase(hp_hbm, cs, cd, buf_a, buf_b, acc, sem_a, sem_b):
    pltpu.async_copy(hp_hbm.at[cs.at[0]], buf_a, sem_a)
    pltpu.async_copy(hp_hbm.at[cs.at[1]], buf_b, sem_b)

    @pl.loop(0, CPP // 2)
    def _(p):
        c = p * 2
        pltpu.make_async_copy(hp_hbm.at[cs.at[c]], buf_a, sem_a).wait()
        pltpu.sync_copy(buf_a, acc.at[cd.at[c]], add=True)

        @pl.when(p < CPP // 2 - 1)
        def _():
            pltpu.async_copy(hp_hbm.at[cs.at[c + 2]], buf_a, sem_a)

        pltpu.make_async_copy(hp_hbm.at[cs.at[c + 1]], buf_b, sem_b).wait()
        pltpu.sync_copy(buf_b, acc.at[cd.at[c + 1]], add=True)

        @pl.when(p < CPP // 2 - 1)
        def _():
            pltpu.async_copy(hp_hbm.at[cs.at[c + 3]], buf_b, sem_b)


@functools.partial(
    pl.kernel,
    mesh=_mesh,
    out_type=jax.ShapeDtypeStruct((NC * N_PAD, D), jnp.float32),
    scratch_types=[
        pltpu.VMEM((CPP, CHUNK), jnp.int32),
        pltpu.VMEM((CPP, CHUNK), jnp.int32),
        pltpu.VMEM((CPP, CHUNK), jnp.int32),
        pltpu.VMEM((CPP, CHUNK), jnp.int32),
        pltpu.VMEM((CHUNK, D), jnp.float32),
        pltpu.VMEM((CHUNK, D), jnp.float32),
        pltpu.VMEM((16, D), jnp.float32),
        pltpu.VMEM_SHARED((N_PAD, D), jnp.float32),
        pltpu.SemaphoreType.DMA,
        pltpu.SemaphoreType.DMA,
        pltpu.SemaphoreType.DMA,
        pltpu.SemaphoreType.DMA,
    ],
)
def _sc_adj(hp_hbm, src_hbm, dst_hbm, out_hbm, idx_s0, idx_s1, idx_d0,
            idx_d1, buf_a, buf_b, zb, acc, sem_i, sem_a, sem_b, sem_z):
    cid = lax.axis_index("c")
    sid = lax.axis_index("s")
    wid = cid * NS + sid

    ic_s = pltpu.async_copy(src_hbm.at[wid * PH], idx_s0, sem_i)
    ic_d = pltpu.async_copy(dst_hbm.at[wid * PH], idx_d0, sem_i)

    @pl.loop(0, 16)
    def _(r):
        @pl.loop(0, D, step=16)
        def _(j):
            zb[r, pl.ds(j, 16)] = jnp.zeros((16,), jnp.float32)

    @pl.loop(0, RPT // 16)
    def _(b):
        pltpu.async_copy(zb, acc.at[pl.ds(sid * RPT + b * 16, 16)], sem_z)

    @pl.loop(0, RPT // 16)
    def _(b):
        pltpu.make_async_copy(zb, acc.at[pl.ds(sid * RPT + b * 16, 16)],
                              sem_z).wait()

    ic_s.wait()
    ic_d.wait()
    plsc.subcore_barrier()

    idx = [(idx_s0, idx_d0), (idx_s1, idx_d1)]
    for ph in range(PH):
        cs, cd = idx[ph % 2]
        ns, nd = idx[(ph + 1) % 2]
        if ph < PH - 1:
            pltpu.async_copy(src_hbm.at[wid * PH + ph + 1], ns, sem_i)
            pltpu.async_copy(dst_hbm.at[wid * PH + ph + 1], nd, sem_i)
        _adj_phase(hp_hbm, cs, cd, buf_a, buf_b, acc, sem_a, sem_b)
        if ph < PH - 1:
            pltpu.make_async_copy(src_hbm.at[wid * PH + ph + 1], ns,
                                  sem_i).wait()
            pltpu.make_async_copy(dst_hbm.at[wid * PH + ph + 1], nd,
                                  sem_i).wait()

    plsc.subcore_barrier()
    pltpu.sync_copy(
        acc.at[pl.ds(sid * RPT, RPT)],
        out_hbm.at[pl.ds(cid * N_PAD + sid * RPT, RPT)],
    )


_BLK = 256
_GRID = N_PAD // _BLK


def _tc_prep_body(d0_ref, d1_ref, x_ref, dis_ref, hp_ref):
    i = pl.program_id(0)
    row = i * _BLK + lax.broadcasted_iota(jnp.int32, (_BLK, 1), 0)
    deg = d0_ref[:, 0:1] + d1_ref[:, 0:1]
    dis = jnp.where(deg > 0, lax.rsqrt(jnp.where(deg > 0, deg, 1.0)), 0.0)
    dis = jnp.where(row < N, dis, 0.0)
    dis_b = jnp.broadcast_to(dis, (_BLK, D))
    dis_ref[...] = dis_b
    hp_ref[...] = dis_b * x_ref[...]


def _tc_prep(d0, d1, x_pad):
    return pl.pallas_call(
        _tc_prep_body,
        grid=(_GRID,),
        in_specs=[
            pl.BlockSpec((_BLK, DEG_W), lambda i: (i, 0)),
            pl.BlockSpec((_BLK, DEG_W), lambda i: (i, 0)),
            pl.BlockSpec((_BLK, D), lambda i: (i, 0)),
        ],
        out_specs=[
            pl.BlockSpec((_BLK, D), lambda i: (i, 0)),
            pl.BlockSpec((_BLK, D), lambda i: (i, 0)),
        ],
        out_shape=[
            jax.ShapeDtypeStruct((N_PAD, D), jnp.float32),
            jax.ShapeDtypeStruct((N_PAD, D), jnp.float32),
        ],
    )(d0, d1, x_pad)


def _tc_step_body(h_ref, p0_ref, p1_ref, dis_ref, w0_ref, w1_ref,
                  hn_ref, hpn_ref):
    hp = dis_ref[...] * (p0_ref[...] + p1_ref[...])
    hn = jnp.dot(h_ref[...], w0_ref[...],
                 preferred_element_type=jnp.float32,
                 precision=lax.Precision.HIGHEST)
    hn += jnp.dot(hp, w1_ref[...],
                  preferred_element_type=jnp.float32,
                  precision=lax.Precision.HIGHEST)
    hn_ref[...] = hn
    hpn_ref[...] = dis_ref[...] * hn


def _tc_step(h, p0, p1, dis_b, w0, w1):
    return pl.pallas_call(
        _tc_step_body,
        grid=(_GRID,),
        in_specs=[
            pl.BlockSpec((_BLK, D), lambda i: (i, 0)),
            pl.BlockSpec((_BLK, D), lambda i: (i, 0)),
            pl.BlockSpec((_BLK, D), lambda i: (i, 0)),
            pl.BlockSpec((_BLK, D), lambda i: (i, 0)),
            pl.BlockSpec((D, D), lambda i: (0, 0)),
            pl.BlockSpec((D, D), lambda i: (0, 0)),
        ],
        out_specs=[
            pl.BlockSpec((_BLK, D), lambda i: (i, 0)),
            pl.BlockSpec((_BLK, D), lambda i: (i, 0)),
        ],
        out_shape=[
            jax.ShapeDtypeStruct((N_PAD, D), jnp.float32),
            jax.ShapeDtypeStruct((N_PAD, D), jnp.float32),
        ],
    )(h, p0, p1, dis_b, w0, w1)


def kernel(x, edge_index, W):
    x_pad = jnp.pad(x, ((0, N_PAD - N), (0, 0)))
    src = jnp.pad(edge_index[0], (0, E_PAD - E), constant_values=PAD_ROW)
    dst = jnp.pad(edge_index[1], (0, E_PAD - E), constant_values=PAD_ROW)
    src4 = src.reshape(NW * PH, CPP, CHUNK)
    dst4 = dst.reshape(NW * PH, CPP, CHUNK)
    dst3 = dst.reshape(NW, NCHUNK, CHUNK)

    deg = _sc_degree(dst3)
    dis_b, hp = _tc_prep(deg[:N_PAD], deg[N_PAD:], x_pad)

    h = x_pad
    for s in range(STEPS):
        parts = _sc_adj(hp, src4, dst4)
        h, hp = _tc_step(h, parts[:N_PAD], parts[N_PAD:], dis_b,
                         W[s, 0], W[s, 1])
    return h[:N]


# speedup vs baseline: 1.0473x; 1.0079x over previous
import functools

import jax
import jax.numpy as jnp
from jax import lax
from jax.experimental import pallas as pl
from jax.experimental.pallas import tpu as pltpu
from jax.experimental.pallas import tpu_sc as plsc

N = 10000
D = 128
E = 320000
STEPS = 4

N_PAD = 10240
PAD_ROW = N
NC, NS = 2, 16
NW = NC * NS
E_PAD = 327680
EPT = E_PAD // NW
CHUNK = 128
NCHUNK = EPT // CHUNK
RPT = N_PAD // NS
DEG_W = 128

_mesh = plsc.VectorSubcoreMesh(core_axis_name="c", subcore_axis_name="s")


@functools.partial(
    pl.kernel,
    mesh=_mesh,
    out_type=jax.ShapeDtypeStruct((NC * N_PAD, DEG_W), jnp.float32),
    scratch_types=[
        pltpu.VMEM((NCHUNK, CHUNK), jnp.int32),
        pltpu.VMEM((CHUNK, DEG_W), jnp.float32),
        pltpu.VMEM((16, DEG_W), jnp.float32),
        pltpu.VMEM_SHARED((N_PAD, DEG_W), jnp.float32),
        pltpu.SemaphoreType.DMA,
    ],
)
def _sc_degree(dst_hbm, out_hbm, idx_d, ones_v, zb, acc, sem):
    cid = lax.axis_index("c")
    sid = lax.axis_index("s")
    wid = cid * NS + sid

    @pl.loop(0, CHUNK)
    def _(r):
        @pl.loop(0, DEG_W, step=16)
        def _(j):
            ones_v[r, pl.ds(j, 16)] = jnp.ones((16,), jnp.float32)

    @pl.loop(0, 16)
    def _(r):
        @pl.loop(0, DEG_W, step=16)
        def _(j):
            zb[r, pl.ds(j, 16)] = jnp.zeros((16,), jnp.float32)

    @pl.loop(0, RPT // 16)
    def _(b):
        pltpu.sync_copy(zb, acc.at[pl.ds(sid * RPT + b * 16, 16)])

    pltpu.async_copy(dst_hbm.at[wid], idx_d, sem).wait()
    plsc.subcore_barrier()

    @pl.loop(0, NCHUNK)
    def _(c):
        pltpu.sync_copy(ones_v, acc.at[idx_d.at[c]], add=True)

    plsc.subcore_barrier()
    pltpu.sync_copy(
        acc.at[pl.ds(sid * RPT, RPT)],
        out_hbm.at[pl.ds(cid * N_PAD + sid * RPT, RPT)],
    )


PH = 4
CPP = NCHUNK // PH


def _adj_phase(hp_hbm, cs, cd, buf_a, buf_b, acc, sem_a, sem_b):
    pltpu.async_copy(hp_hbm.at[cs.at[0]], buf_a, sem_a)
    pltpu.async_copy(hp_hbm.at[cs.at[1]], buf_b, sem_b)

    @pl.loop(0, CPP // 2)
    def _(p):
        c = p * 2
        pltpu.make_async_copy(hp_hbm.at[cs.at[c]], buf_a, sem_a).wait()
        pltpu.sync_copy(buf_a, acc.at[cd.at[c]], add=True)

        @pl.when(p < CPP // 2 - 1)
        def _():
            pltpu.async_copy(hp_hbm.at[cs.at[c + 2]], buf_a, sem_a)

        pltpu.make_async_copy(hp_hbm.at[cs.at[c + 1]], buf_b, sem_b).wait()
        pltpu.sync_copy(buf_b, acc.at[cd.at[c + 1]], add=True)

        @pl.when(p < CPP // 2 - 1)
        def _():
            pltpu.async_copy(hp_hbm.at[cs.at[c + 3]], buf_b, sem_b)


@functools.partial(
    pl.kernel,
    mesh=_mesh,
    out_type=jax.ShapeDtypeStruct((NC * N_PAD, D), jnp.float32),
    scratch_types=[
        pltpu.VMEM((CPP, CHUNK), jnp.int32),
        pltpu.VMEM((CPP, CHUNK), jnp.int32),
        pltpu.VMEM((CPP, CHUNK), jnp.int32),
        pltpu.VMEM((CPP, CHUNK), jnp.int32),
        pltpu.VMEM((CHUNK, D), jnp.float32),
        pltpu.VMEM((CHUNK, D), jnp.float32),
        pltpu.VMEM((16, D), jnp.float32),
        pltpu.VMEM_SHARED((N_PAD, D), jnp.float32),
        pltpu.SemaphoreType.DMA,
        pltpu.SemaphoreType.DMA,
        pltpu.SemaphoreType.DMA,
        pltpu.SemaphoreType.DMA,
    ],
)
def _sc_adj(hp_hbm, src_hbm, dst_hbm, out_hbm, idx_s0, idx_s1, idx_d0,
            idx_d1, buf_a, buf_b, zb, acc, sem_i, sem_a, sem_b, sem_z):
    cid = lax.axis_index("c")
    sid = lax.axis_index("s")
    wid = cid * NS + sid

    ic_s = pltpu.async_copy(src_hbm.at[wid * PH], idx_s0, sem_i)
    ic_d = pltpu.async_copy(dst_hbm.at[wid * PH], idx_d0, sem_i)

    @pl.loop(0, 16)
    def _(r):
        @pl.loop(0, D, step=16)
        def _(j):
            zb[r, pl.ds(j, 16)] = jnp.zeros((16,), jnp.float32)

    @pl.loop(0, RPT // 16)
    def _(b):
        pltpu.async_copy(zb, acc.at[pl.ds(sid * RPT + b * 16, 16)], sem_z)

    @pl.loop(0, RPT // 16)
    def _(b):
        pltpu.make_async_copy(zb, acc.at[pl.ds(sid * RPT + b * 16, 16)],
                              sem_z).wait()

    ic_s.wait()
    ic_d.wait()
    plsc.subcore_barrier()

    idx = [(idx_s0, idx_d0), (idx_s1, idx_d1)]
    for ph in range(PH):
        cs, cd = idx[ph % 2]
        ns, nd = idx[(ph + 1) % 2]
        if ph < PH - 1:
            pltpu.async_copy(src_hbm.at[wid * PH + ph + 1], ns, sem_i)
            pltpu.async_copy(dst_hbm.at[wid * PH + ph + 1], nd, sem_i)
        _adj_phase(hp_hbm, cs, cd, buf_a, buf_b, acc, sem_a, sem_b)
        if ph < PH - 1:
            pltpu.make_async_copy(src_hbm.at[wid * PH + ph + 1], ns,
                                  sem_i).wait()
            pltpu.make_async_copy(dst_hbm.at[wid * PH + ph + 1], nd,
                                  sem_i).wait()

    plsc.subcore_barrier()
    pltpu.sync_copy(
        acc.at[pl.ds(sid * RPT, RPT)],
        out_hbm.at[pl.ds(cid * N_PAD + sid * RPT, RPT)],
    )


_BLK = 256
_GRID = N_PAD // _BLK


def _tc_prep_body(d0_ref, d1_ref, x_ref, dis_ref, hp_ref):
    i = pl.program_id(0)
    row = i * _BLK + lax.broadcasted_iota(jnp.int32, (_BLK, 1), 0)
    deg = d0_ref[:, 0:1] + d1_ref[:, 0:1]
    dis = jnp.where(deg > 0, lax.rsqrt(jnp.where(deg > 0, deg, 1.0)), 0.0)
    dis = jnp.where(row < N, dis, 0.0)
    dis_b = jnp.broadcast_to(dis, (_BLK, D))
    dis_ref[...] = dis_b
    hp_ref[...] = dis_b * x_ref[...]


def _tc_prep(d0, d1, x_pad):
    return pl.pallas_call(
        _tc_prep_body,
        grid=(_GRID,),
        in_specs=[
            pl.BlockSpec((_BLK, DEG_W), lambda i: (i, 0)),
            pl.BlockSpec((_BLK, DEG_W), lambda i: (i, 0)),
            pl.BlockSpec((_BLK, D), lambda i: (i, 0)),
        ],
        out_specs=[
            pl.BlockSpec((_BLK, D), lambda i: (i, 0)),
            pl.BlockSpec((_BLK, D), lambda i: (i, 0)),
        ],
        out_shape=[
            jax.ShapeDtypeStruct((N_PAD, D), jnp.float32),
            jax.ShapeDtypeStruct((N_PAD, D), jnp.float32),
        ],
    )(d0, d1, x_pad)


def _tc_step_body(h_ref, p0_ref, p1_ref, dis_ref, w0_ref, w1_ref,
                  hn_ref, hpn_ref):
    hp = dis_ref[...] * (p0_ref[...] + p1_ref[...])
    hn = jnp.dot(h_ref[...], w0_ref[...],
                 preferred_element_type=jnp.float32)
    hn += jnp.dot(hp, w1_ref[...],
                  preferred_element_type=jnp.float32)
    hn_ref[...] = hn
    hpn_ref[...] = dis_ref[...] * hn


def _tc_step(h, p0, p1, dis_b, w0, w1):
    return pl.pallas_call(
        _tc_step_body,
        grid=(_GRID,),
        in_specs=[
            pl.BlockSpec((_BLK, D), lambda i: (i, 0)),
            pl.BlockSpec((_BLK, D), lambda i: (i, 0)),
            pl.BlockSpec((_BLK, D), lambda i: (i, 0)),
            pl.BlockSpec((_BLK, D), lambda i: (i, 0)),
            pl.BlockSpec((D, D), lambda i: (0, 0)),
            pl.BlockSpec((D, D), lambda i: (0, 0)),
        ],
        out_specs=[
            pl.BlockSpec((_BLK, D), lambda i: (i, 0)),
            pl.BlockSpec((_BLK, D), lambda i: (i, 0)),
        ],
        out_shape=[
            jax.ShapeDtypeStruct((N_PAD, D), jnp.float32),
            jax.ShapeDtypeStruct((N_PAD, D), jnp.float32),
        ],
    )(h, p0, p1, dis_b, w0, w1)


def kernel(x, edge_index, W):
    x_pad = jnp.pad(x, ((0, N_PAD - N), (0, 0)))
    src = jnp.pad(edge_index[0], (0, E_PAD - E), constant_values=PAD_ROW)
    dst = jnp.pad(edge_index[1], (0, E_PAD - E), constant_values=PAD_ROW)
    src4 = src.reshape(NW * PH, CPP, CHUNK)
    dst4 = dst.reshape(NW * PH, CPP, CHUNK)
    dst3 = dst.reshape(NW, NCHUNK, CHUNK)

    deg = _sc_degree(dst3)
    dis_b, hp = _tc_prep(deg[:N_PAD], deg[N_PAD:], x_pad)

    h = x_pad
    for s in range(STEPS):
        parts = _sc_adj(hp, src4, dst4)
        h, hp = _tc_step(h, parts[:N_PAD], parts[N_PAD:], dis_b,
                         W[s, 0], W[s, 1])
    return h[:N]
